# Initial kernel scaffold; baseline (speedup 1.0000x reference)
import jax
import jax.numpy as jnp
from jax.experimental import pallas as pl


def kernel(x, edge_index, W1, b1, W2, b2, fcW, fcb):
    raise NotImplementedError("write your pallas kernel here")


# speedup vs baseline: 29.7857x; 29.7857x over previous
import functools

import jax
import jax.numpy as jnp
from jax import lax
from jax.experimental import pallas as pl
from jax.experimental.pallas import tpu as pltpu
from jax.experimental.pallas import tpu_sc as plsc

N_NODES = 100000
N_EDGES = 6400000
LANES = 128
N_ROWS = N_EDGES // LANES
G = 8
N_GROUPS = N_ROWS // G
NC = 2
NS = 16
NW = NC * NS
R_A = 6256
R_LAST = N_NODES - (NS - 1) * R_A


def _striped(sid, copy_fn):
    r0 = pl.multiple_of(sid * R_A, 8)

    @pl.when(sid < NS - 1)
    def _():
        copy_fn(r0, R_A)

    @pl.when(sid == NS - 1)
    def _():
        copy_fn((NS - 1) * R_A, R_LAST)


def _sc_mesh():
    return plsc.VectorSubcoreMesh(core_axis_name="c", subcore_axis_name="s")


_SC_PARAMS = pltpu.CompilerParams(use_tc_tiling_on_sc=False)


def _make_agg_kernel(width):
    @functools.partial(
        pl.kernel,
        out_type=jax.ShapeDtypeStruct((NC, N_NODES, width), jnp.float32),
        mesh=_sc_mesh(),
        scratch_types=[
            [pltpu.VMEM((LANES,), jnp.int32) for _ in range(G)],
            [pltpu.VMEM((LANES,), jnp.int32) for _ in range(G)],
            pltpu.VMEM((G, LANES, width), jnp.float32),
            pltpu.VMEM_SHARED((N_NODES, width), jnp.float32),
            pltpu.SemaphoreType.DMA,
        ],
        compiler_params=_SC_PARAMS,
    )
    def agg(g_hbm, src_hbm, dst_hbm, z_hbm, out_hbm,
            sidx1, didx1, rows, acc, sem):
        cid = lax.axis_index("c")
        sid = lax.axis_index("s")
        w = sid * NC + cid

        _striped(sid, lambda o, n: pltpu.sync_copy(
            z_hbm.at[pl.ds(o, n)], acc.at[pl.ds(o, n)]))
        plsc.subcore_barrier()

        def group(i, carry):
            t = w + i * NW
            @pl.when(t < N_GROUPS)
            def _():
                e0 = pl.multiple_of(t * (G * LANES), 8)
                for j in range(G):
                    pltpu.sync_copy(src_hbm.at[pl.ds(e0 + j * LANES, LANES)],
                                    sidx1[j])
                    pltpu.sync_copy(dst_hbm.at[pl.ds(e0 + j * LANES, LANES)],
                                    didx1[j])
                for j in range(G):
                    pltpu.async_copy(g_hbm.at[sidx1[j]], rows.at[j], sem)
                for j in range(G):
                    pltpu.make_async_copy(g_hbm.at[sidx1[j]], rows.at[j],
                                          sem).wait()
                for j in range(G):
                    pltpu.sync_copy(rows.at[j], acc.at[didx1[j]], add=True)
            return carry

        n_iter = (N_GROUPS - 1) // NW + 1
        lax.fori_loop(0, n_iter, group, 0)

        plsc.subcore_barrier()
        _striped(sid, lambda o, n: pltpu.sync_copy(
            acc.at[pl.ds(o, n)], out_hbm.at[cid, pl.ds(o, n)]))

    return agg


def _make_deg_kernel():
    @functools.partial(
        pl.kernel,
        out_type=jax.ShapeDtypeStruct((NC, N_NODES, 1), jnp.float32),
        mesh=_sc_mesh(),
        scratch_types=[
            [pltpu.VMEM((LANES,), jnp.int32) for _ in range(G)],
            pltpu.VMEM((G, LANES, 1), jnp.float32),
            pltpu.VMEM_SHARED((N_NODES, 1), jnp.float32),
        ],
        compiler_params=_SC_PARAMS,
    )
    def deg(dst_hbm, z_hbm, o_hbm, out_hbm, didx1, ones, acc):
        cid = lax.axis_index("c")
        sid = lax.axis_index("s")
        w = sid * NC + cid

        pltpu.sync_copy(o_hbm, ones)

        _striped(sid, lambda o, n: pltpu.sync_copy(
            z_hbm.at[pl.ds(o, n)], acc.at[pl.ds(o, n)]))
        plsc.subcore_barrier()

        def group(i, carry):
            t = w + i * NW
            @pl.when(t < N_GROUPS)
            def _():
                e0 = pl.multiple_of(t * (G * LANES), 8)
                for j in range(G):
                    pltpu.sync_copy(dst_hbm.at[pl.ds(e0 + j * LANES, LANES)],
                                    didx1[j])
                for j in range(G):
                    pltpu.sync_copy(ones.at[j], acc.at[didx1[j]], add=True)
            return carry

        n_iter = (N_GROUPS - 1) // NW + 1
        lax.fori_loop(0, n_iter, group, 0)

        plsc.subcore_barrier()
        _striped(sid, lambda o, n: pltpu.sync_copy(
            acc.at[pl.ds(o, n)], out_hbm.at[cid, pl.ds(o, n)]))

    return deg


_BLK = 4000
_GRID = N_NODES // _BLK


def _d1_body(degp, x, W1, g1, dis):
    d = degp[0] + degp[1] + 1.0
    r = lax.rsqrt(d)
    h = jnp.dot(x[...], W1[...], preferred_element_type=jnp.float32)
    g1[...] = h * r
    dis[...] = r


def _d2_body(s1, g1, dis, W2, b1, g2):
    r = dis[...]
    agg = (s1[0] + s1[1] + g1[...]) * r
    h1 = jnp.maximum(agg + b1[...], 0.0)
    g2[...] = jnp.dot(h1, W2[...], preferred_element_type=jnp.float32) * r


def _d3_body(s2, g2, dis, b2, fcW, fcb, out):
    r = dis[...]
    h2 = jnp.maximum((s2[0] + s2[1] + g2[...]) * r + b2[...], 0.0)
    z = jnp.dot(h2, fcW[...], preferred_element_type=jnp.float32) + fcb[...]
    out[...] = jax.nn.sigmoid(z)


def _row_spec(width):
    return pl.BlockSpec((_BLK, width), lambda i: (i, 0))


def _part_spec(width):
    return pl.BlockSpec((NC, _BLK, width), lambda i: (0, i, 0))


def _full_spec(a, b):
    return pl.BlockSpec((a, b), lambda i: (0, 0))


def kernel(x, edge_index, W1, b1, W2, b2, fcW, fcb):
    src = edge_index[0].astype(jnp.int32)
    dst = edge_index[1].astype(jnp.int32)
    z1 = jnp.zeros((N_NODES, 1), jnp.float32)
    z16 = jnp.zeros((N_NODES, 16), jnp.float32)
    z8 = jnp.zeros((N_NODES, 8), jnp.float32)

    ones_rows = jnp.ones((G, LANES, 1), jnp.float32)
    degp = _make_deg_kernel()(dst, z1, ones_rows)

    d1 = pl.pallas_call(
        _d1_body,
        grid=(_GRID,),
        in_specs=[_part_spec(1), _row_spec(12), _full_spec(12, 16)],
        out_specs=[_row_spec(16), _row_spec(1)],
        out_shape=[
            jax.ShapeDtypeStruct((N_NODES, 16), jnp.float32),
            jax.ShapeDtypeStruct((N_NODES, 1), jnp.float32),
        ],
    )
    g1, dis = d1(degp, x, W1)

    s1 = _make_agg_kernel(16)(g1, src, dst, z16)

    d2 = pl.pallas_call(
        _d2_body,
        grid=(_GRID,),
        in_specs=[_part_spec(16), _row_spec(16), _row_spec(1),
                  _full_spec(16, 8), _full_spec(1, 16)],
        out_specs=_row_spec(8),
        out_shape=jax.ShapeDtypeStruct((N_NODES, 8), jnp.float32),
    )
    g2 = d2(s1, g1, dis, W2, b1.reshape(1, 16))

    s2 = _make_agg_kernel(8)(g2, src, dst, z8)

    d3 = pl.pallas_call(
        _d3_body,
        grid=(_GRID,),
        in_specs=[_part_spec(8), _row_spec(8), _row_spec(1),
                  _full_spec(1, 8), _full_spec(8, 1), _full_spec(1, 1)],
        out_specs=_row_spec(1),
        out_shape=jax.ShapeDtypeStruct((N_NODES, 1), jnp.float32),
    )
    out = d3(s2, g2, dis, b2.reshape(1, 8), fcW, fcb.reshape(1, 1))
    return out


# speedup vs baseline: 72.0955x; 2.4205x over previous
import functools

import jax
import jax.numpy as jnp
from jax import lax
from jax.experimental import pallas as pl
from jax.experimental.pallas import tpu as pltpu
from jax.experimental.pallas import tpu_sc as plsc

N_NODES = 100000
N_EDGES = 6400000
LANES = 128
N_ROWS = N_EDGES // LANES
G = 8
N_GROUPS = N_ROWS // G
NC = 2
NS = 16
NW = NC * NS
R_A = 6256
R_LAST = N_NODES - (NS - 1) * R_A


def _striped(sid, copy_fn):
    r0 = pl.multiple_of(sid * R_A, 8)

    @pl.when(sid < NS - 1)
    def _():
        copy_fn(r0, R_A)

    @pl.when(sid == NS - 1)
    def _():
        copy_fn((NS - 1) * R_A, R_LAST)


def _sc_mesh():
    return plsc.VectorSubcoreMesh(core_axis_name="c", subcore_axis_name="s")


_SC_PARAMS = pltpu.CompilerParams(use_tc_tiling_on_sc=False)


def _make_agg_kernel(width):
    @functools.partial(
        pl.kernel,
        out_type=jax.ShapeDtypeStruct((NC, N_NODES, width), jnp.float32),
        mesh=_sc_mesh(),
        scratch_types=[
            [pltpu.VMEM((LANES,), jnp.int32) for _ in range(G)],
            [pltpu.VMEM((LANES,), jnp.int32) for _ in range(G)],
            pltpu.VMEM((G, LANES, width), jnp.float32),
            pltpu.VMEM_SHARED((N_NODES, width), jnp.float32),
            pltpu.SemaphoreType.DMA,
            pltpu.SemaphoreType.DMA,
        ],
        compiler_params=_SC_PARAMS,
    )
    def agg(g_hbm, src_hbm, dst_hbm, z_hbm, out_hbm,
            sidx1, didx1, rows, acc, sem, semi):
        cid = lax.axis_index("c")
        sid = lax.axis_index("s")
        w = sid * NC + cid

        _striped(sid, lambda o, n: pltpu.sync_copy(
            z_hbm.at[pl.ds(o, n)], acc.at[pl.ds(o, n)]))
        plsc.subcore_barrier()

        def group(i, carry):
            t = w + i * NW
            @pl.when(t < N_GROUPS)
            def _():
                e0 = pl.multiple_of(t * (G * LANES), 8)
                idx_dmas = []
                for j in range(G):
                    idx_dmas.append(pltpu.async_copy(
                        src_hbm.at[pl.ds(e0 + j * LANES, LANES)],
                        sidx1[j], semi))
                    idx_dmas.append(pltpu.async_copy(
                        dst_hbm.at[pl.ds(e0 + j * LANES, LANES)],
                        didx1[j], semi))
                for d in idx_dmas:
                    d.wait()
                gats = [pltpu.async_copy(
                    g_hbm.at[sidx1[j]], rows.at[j], sem) for j in range(G)]
                for d in gats:
                    d.wait()
                for j in range(G):
                    pltpu.sync_copy(rows.at[j], acc.at[didx1[j]], add=True)
            return carry

        n_iter = (N_GROUPS - 1) // NW + 1
        lax.fori_loop(0, n_iter, group, 0)

        plsc.subcore_barrier()
        _striped(sid, lambda o, n: pltpu.sync_copy(
            acc.at[pl.ds(o, n)], out_hbm.at[cid, pl.ds(o, n)]))

    return agg


def _make_deg_kernel():
    @functools.partial(
        pl.kernel,
        out_type=jax.ShapeDtypeStruct((NC, N_NODES, 1), jnp.float32),
        mesh=_sc_mesh(),
        scratch_types=[
            [pltpu.VMEM((LANES,), jnp.int32) for _ in range(G)],
            pltpu.VMEM((G, LANES, 1), jnp.float32),
            pltpu.VMEM_SHARED((N_NODES, 1), jnp.float32),
            pltpu.SemaphoreType.DMA,
        ],
        compiler_params=_SC_PARAMS,
    )
    def deg(dst_hbm, z_hbm, o_hbm, out_hbm, didx1, ones, acc, semi):
        cid = lax.axis_index("c")
        sid = lax.axis_index("s")
        w = sid * NC + cid

        pltpu.sync_copy(o_hbm, ones)

        _striped(sid, lambda o, n: pltpu.sync_copy(
            z_hbm.at[pl.ds(o, n)], acc.at[pl.ds(o, n)]))
        plsc.subcore_barrier()

        def group(i, carry):
            t = w + i * NW
            @pl.when(t < N_GROUPS)
            def _():
                e0 = pl.multiple_of(t * (G * LANES), 8)
                idx_dmas = [pltpu.async_copy(
                    dst_hbm.at[pl.ds(e0 + j * LANES, LANES)],
                    didx1[j], semi) for j in range(G)]
                for d in idx_dmas:
                    d.wait()
                for j in range(G):
                    pltpu.sync_copy(ones.at[j], acc.at[didx1[j]], add=True)
            return carry

        n_iter = (N_GROUPS - 1) // NW + 1
        lax.fori_loop(0, n_iter, group, 0)

        plsc.subcore_barrier()
        _striped(sid, lambda o, n: pltpu.sync_copy(
            acc.at[pl.ds(o, n)], out_hbm.at[cid, pl.ds(o, n)]))

    return deg


_BLK = 4000
_GRID = N_NODES // _BLK


def _d1_body(degp, x, W1, g1, dis):
    d = degp[0] + degp[1] + 1.0
    r = lax.rsqrt(d)
    h = jnp.dot(x[...], W1[...], preferred_element_type=jnp.float32)
    g1[...] = h * r
    dis[...] = r


def _d2_body(s1, g1, dis, W2, b1, g2):
    r = dis[...]
    agg = (s1[0] + s1[1] + g1[...]) * r
    h1 = jnp.maximum(agg + b1[...], 0.0)
    g2[...] = jnp.dot(h1, W2[...], preferred_element_type=jnp.float32) * r


def _d3_body(s2, g2, dis, b2, fcW, fcb, out):
    r = dis[...]
    h2 = jnp.maximum((s2[0] + s2[1] + g2[...]) * r + b2[...], 0.0)
    z = jnp.dot(h2, fcW[...], preferred_element_type=jnp.float32) + fcb[...]
    out[...] = jax.nn.sigmoid(z)


def _row_spec(width):
    return pl.BlockSpec((_BLK, width), lambda i: (i, 0))


def _part_spec(width):
    return pl.BlockSpec((NC, _BLK, width), lambda i: (0, i, 0))


def _full_spec(a, b):
    return pl.BlockSpec((a, b), lambda i: (0, 0))


def kernel(x, edge_index, W1, b1, W2, b2, fcW, fcb):
    src = edge_index[0].astype(jnp.int32)
    dst = edge_index[1].astype(jnp.int32)
    z1 = jnp.zeros((N_NODES, 1), jnp.float32)
    z16 = jnp.zeros((N_NODES, 16), jnp.float32)
    z8 = jnp.zeros((N_NODES, 8), jnp.float32)

    ones_rows = jnp.ones((G, LANES, 1), jnp.float32)
    degp = _make_deg_kernel()(dst, z1, ones_rows)

    d1 = pl.pallas_call(
        _d1_body,
        grid=(_GRID,),
        in_specs=[_part_spec(1), _row_spec(12), _full_spec(12, 16)],
        out_specs=[_row_spec(16), _row_spec(1)],
        out_shape=[
            jax.ShapeDtypeStruct((N_NODES, 16), jnp.float32),
            jax.ShapeDtypeStruct((N_NODES, 1), jnp.float32),
        ],
    )
    g1, dis = d1(degp, x, W1)

    s1 = _make_agg_kernel(16)(g1, src, dst, z16)

    d2 = pl.pallas_call(
        _d2_body,
        grid=(_GRID,),
        in_specs=[_part_spec(16), _row_spec(16), _row_spec(1),
                  _full_spec(16, 8), _full_spec(1, 16)],
        out_specs=_row_spec(8),
        out_shape=jax.ShapeDtypeStruct((N_NODES, 8), jnp.float32),
    )
    g2 = d2(s1, g1, dis, W2, b1.reshape(1, 16))

    s2 = _make_agg_kernel(8)(g2, src, dst, z8)

    d3 = pl.pallas_call(
        _d3_body,
        grid=(_GRID,),
        in_specs=[_part_spec(8), _row_spec(8), _row_spec(1),
                  _full_spec(1, 8), _full_spec(8, 1), _full_spec(1, 1)],
        out_specs=_row_spec(1),
        out_shape=jax.ShapeDtypeStruct((N_NODES, 1), jnp.float32),
    )
    out = d3(s2, g2, dis, b2.reshape(1, 8), fcW, fcb.reshape(1, 1))
    return out
